# Initial kernel scaffold; baseline (speedup 1.0000x reference)
import jax
import jax.numpy as jnp
from jax.experimental import pallas as pl


def kernel(hidden_states, weight):
    raise NotImplementedError("write your pallas kernel here")


# speedup vs baseline: 1.0911x; 1.0911x over previous
import functools

import jax
import jax.numpy as jnp
from jax.experimental import pallas as pl

_TOP_K = 8


def _router_block(x_ref, w_ref, probs_ref, tw_ref, ti_ref, *, n_experts):
    x = x_ref[...]
    w = w_ref[...]
    logits = jax.lax.dot_general(
        x, w, (((1,), (1,)), ((), ())), preferred_element_type=jnp.float32
    )
    m = jnp.max(logits, axis=-1, keepdims=True)
    e = jnp.exp(logits - m)
    s = jnp.sum(e, axis=-1, keepdims=True)
    probs = e / s
    probs_ref[...] = probs

    cols = jax.lax.broadcasted_iota(jnp.int32, probs.shape, 1)
    work = probs
    vals = []
    idxs = []
    for _ in range(_TOP_K):
        mk = jnp.max(work, axis=-1, keepdims=True)
        is_max = work == mk
        cand = jnp.where(is_max, cols, n_experts)
        ik = jnp.min(cand, axis=-1, keepdims=True)
        vals.append(mk)
        idxs.append(ik)
        work = jnp.where(cols == ik, -1.0, work)
    tv = jnp.concatenate(vals, axis=-1)
    ti = jnp.concatenate(idxs, axis=-1)
    tw_ref[...] = tv / (jnp.sum(tv, axis=-1, keepdims=True) + 1e-6)
    ti_ref[...] = ti


@functools.partial(jax.jit, static_argnames=("block_t", "interpret"))
def _router(hidden_states, weight, block_t=512, interpret=False):
    t, d = hidden_states.shape
    n_experts = weight.shape[0]
    grid = (t // block_t,)
    return pl.pallas_call(
        functools.partial(_router_block, n_experts=n_experts),
        grid=grid,
        in_specs=[
            pl.BlockSpec((block_t, d), lambda i: (i, 0)),
            pl.BlockSpec((n_experts, d), lambda i: (0, 0)),
        ],
        out_specs=[
            pl.BlockSpec((block_t, n_experts), lambda i: (i, 0)),
            pl.BlockSpec((block_t, _TOP_K), lambda i: (i, 0)),
            pl.BlockSpec((block_t, _TOP_K), lambda i: (i, 0)),
        ],
        out_shape=[
            jax.ShapeDtypeStruct((t, n_experts), jnp.float32),
            jax.ShapeDtypeStruct((t, _TOP_K), hidden_states.dtype),
            jax.ShapeDtypeStruct((t, _TOP_K), jnp.int32),
        ],
        interpret=interpret,
    )(hidden_states, weight)


def kernel(hidden_states, weight):
    probs, tw, ti = _router(hidden_states, weight)
    return probs, tw, ti


# speedup vs baseline: 1.2105x; 1.1095x over previous
import functools

import jax
import jax.numpy as jnp
from jax.experimental import pallas as pl

_TOP_K = 8


def _router_block(x_ref, w_ref, probs_ref, tw_ref, ti_ref, *, n_experts):
    x = x_ref[...]
    w = w_ref[...]
    logits = jax.lax.dot_general(
        x, w, (((1,), (1,)), ((), ())), preferred_element_type=jnp.float32
    )
    m = jnp.max(logits, axis=-1, keepdims=True)
    e = jnp.exp(logits - m)
    s = jnp.sum(e, axis=-1, keepdims=True)
    probs = e / s
    probs_ref[...] = probs

    fcols = jax.lax.broadcasted_iota(jnp.int32, probs.shape, 1).astype(jnp.float32)
    sentinel = jnp.float32(n_experts)
    work = probs
    vals = []
    idxs = []
    for _ in range(_TOP_K):
        mk = jnp.max(work, axis=-1, keepdims=True)
        cand = jnp.where(work == mk, fcols, sentinel)
        fik = jnp.min(cand, axis=-1, keepdims=True)
        vals.append(mk)
        idxs.append(fik)
        work = jnp.where(cand == fik, -1.0, work)
    tv = jnp.concatenate(vals, axis=-1)
    fti = jnp.concatenate(idxs, axis=-1)
    tw_ref[...] = tv / (jnp.sum(tv, axis=-1, keepdims=True) + 1e-6)
    ti_ref[...] = fti.astype(jnp.int32)


@functools.partial(jax.jit, static_argnames=("block_t", "interpret"))
def _router(hidden_states, weight, block_t=512, interpret=False):
    t, d = hidden_states.shape
    n_experts = weight.shape[0]
    grid = (t // block_t,)
    return pl.pallas_call(
        functools.partial(_router_block, n_experts=n_experts),
        grid=grid,
        in_specs=[
            pl.BlockSpec((block_t, d), lambda i: (i, 0)),
            pl.BlockSpec((n_experts, d), lambda i: (0, 0)),
        ],
        out_specs=[
            pl.BlockSpec((block_t, n_experts), lambda i: (i, 0)),
            pl.BlockSpec((block_t, _TOP_K), lambda i: (i, 0)),
            pl.BlockSpec((block_t, _TOP_K), lambda i: (i, 0)),
        ],
        out_shape=[
            jax.ShapeDtypeStruct((t, n_experts), jnp.float32),
            jax.ShapeDtypeStruct((t, _TOP_K), hidden_states.dtype),
            jax.ShapeDtypeStruct((t, _TOP_K), jnp.int32),
        ],
        interpret=interpret,
    )(hidden_states, weight)


def kernel(hidden_states, weight):
    probs, tw, ti = _router(hidden_states, weight)
    return probs, tw, ti


# speedup vs baseline: 1.2246x; 1.0116x over previous
import functools

import jax
import jax.numpy as jnp
from jax.experimental import pallas as pl

_TOP_K = 8


def _router_block(x_ref, w_ref, probs_ref, tw_ref, ti_ref, *, n_experts):
    x = x_ref[...]
    w = w_ref[...]
    logits = jax.lax.dot_general(
        x, w, (((1,), (1,)), ((), ())), preferred_element_type=jnp.float32
    )
    fcols = jax.lax.broadcasted_iota(jnp.int32, logits.shape, 1).astype(jnp.float32)
    sentinel = jnp.float32(n_experts)
    work = logits
    vals = []
    idxs = []
    for _ in range(_TOP_K):
        mk = jnp.max(work, axis=-1, keepdims=True)
        cand = jnp.where(work == mk, fcols, sentinel)
        fik = jnp.min(cand, axis=-1, keepdims=True)
        vals.append(mk)
        idxs.append(fik)
        work = jnp.where(cand == fik, -jnp.inf, work)

    m = vals[0]
    e = jnp.exp(logits - m)
    s = jnp.sum(e, axis=-1, keepdims=True)
    probs_ref[...] = e / s

    lv = jnp.concatenate(vals, axis=-1)
    fti = jnp.concatenate(idxs, axis=-1)
    ev = jnp.exp(lv - m)
    tw_ref[...] = ev / (jnp.sum(ev, axis=-1, keepdims=True) + 1e-6 * s)
    ti_ref[...] = fti.astype(jnp.int32)


@functools.partial(jax.jit, static_argnames=("block_t", "interpret"))
def _router(hidden_states, weight, block_t=512, interpret=False):
    t, d = hidden_states.shape
    n_experts = weight.shape[0]
    grid = (t // block_t,)
    return pl.pallas_call(
        functools.partial(_router_block, n_experts=n_experts),
        grid=grid,
        in_specs=[
            pl.BlockSpec((block_t, d), lambda i: (i, 0)),
            pl.BlockSpec((n_experts, d), lambda i: (0, 0)),
        ],
        out_specs=[
            pl.BlockSpec((block_t, n_experts), lambda i: (i, 0)),
            pl.BlockSpec((block_t, _TOP_K), lambda i: (i, 0)),
            pl.BlockSpec((block_t, _TOP_K), lambda i: (i, 0)),
        ],
        out_shape=[
            jax.ShapeDtypeStruct((t, n_experts), jnp.float32),
            jax.ShapeDtypeStruct((t, _TOP_K), hidden_states.dtype),
            jax.ShapeDtypeStruct((t, _TOP_K), jnp.int32),
        ],
        interpret=interpret,
    )(hidden_states, weight)


def kernel(hidden_states, weight):
    probs, tw, ti = _router(hidden_states, weight)
    return probs, tw, ti


# speedup vs baseline: 1.3771x; 1.1245x over previous
import functools

import jax
import jax.numpy as jnp
from jax.experimental import pallas as pl
from jax.experimental.pallas import tpu as pltpu

_TOP_K = 8


def _step(x_ref, w_ref, probs_ref, tw_ref, ti_ref, wr_ref, rd_ref, n_experts):
    wr_ref[...] = jax.lax.dot_general(
        x_ref[...], w_ref[...], (((1,), (1,)), ((), ())),
        preferred_element_type=jnp.float32,
    )

    logits = rd_ref[...]
    fcols = jax.lax.broadcasted_iota(
        jnp.int32, logits.shape, 1).astype(jnp.float32)
    sentinel = jnp.float32(n_experts)
    work = logits
    vals = []
    idxs = []
    for _ in range(_TOP_K):
        mk = jnp.max(work, axis=-1, keepdims=True)
        cand = jnp.where(work == mk, fcols, sentinel)
        fik = jnp.min(cand, axis=-1, keepdims=True)
        vals.append(mk)
        idxs.append(fik)
        work = jnp.where(cand == fik, -jnp.inf, work)

    m = vals[0]
    e = jnp.exp(logits - m)
    s = jnp.sum(e, axis=-1, keepdims=True)
    probs_ref[...] = e / s

    lv = jnp.concatenate(vals, axis=-1)
    fti = jnp.concatenate(idxs, axis=-1)
    ev = jnp.exp(lv - m)
    tw_ref[...] = ev / (jnp.sum(ev, axis=-1, keepdims=True) + 1e-6 * s)
    ti_ref[...] = fti.astype(jnp.int32)


def _router_block(x_ref, w_ref, probs_ref, tw_ref, ti_ref, acc_a, acc_b, *,
                  n_experts, n_blocks):
    i = pl.program_id(0)

    @pl.when(i % 2 == 0)
    def _even():
        _step(x_ref, w_ref, probs_ref, tw_ref, ti_ref, acc_a, acc_b,
              n_experts)

    @pl.when(i % 2 == 1)
    def _odd():
        _step(x_ref, w_ref, probs_ref, tw_ref, ti_ref, acc_b, acc_a,
              n_experts)


@functools.partial(jax.jit, static_argnames=("block_t", "interpret"))
def _router(hidden_states, weight, block_t=512, interpret=False):
    t, d = hidden_states.shape
    n_experts = weight.shape[0]
    n_blocks = t // block_t
    return pl.pallas_call(
        functools.partial(_router_block, n_experts=n_experts,
                          n_blocks=n_blocks),
        grid=(n_blocks + 1,),
        in_specs=[
            pl.BlockSpec((block_t, d), lambda i: (jnp.minimum(i, n_blocks - 1), 0)),
            pl.BlockSpec((n_experts, d), lambda i: (0, 0)),
        ],
        out_specs=[
            pl.BlockSpec((block_t, n_experts), lambda i: (jnp.maximum(i - 1, 0), 0)),
            pl.BlockSpec((block_t, _TOP_K), lambda i: (jnp.maximum(i - 1, 0), 0)),
            pl.BlockSpec((block_t, _TOP_K), lambda i: (jnp.maximum(i - 1, 0), 0)),
        ],
        out_shape=[
            jax.ShapeDtypeStruct((t, n_experts), jnp.float32),
            jax.ShapeDtypeStruct((t, _TOP_K), hidden_states.dtype),
            jax.ShapeDtypeStruct((t, _TOP_K), jnp.int32),
        ],
        scratch_shapes=[pltpu.VMEM((block_t, n_experts), jnp.float32),
                        pltpu.VMEM((block_t, n_experts), jnp.float32)],
        interpret=interpret,
    )(hidden_states, weight)


def kernel(hidden_states, weight):
    probs, tw, ti = _router(hidden_states, weight)
    return probs, tw, ti
